# baseline (speedup 1.0000x reference)
import functools

import jax
import jax.numpy as jnp
from jax import lax
from jax.experimental import pallas as pl
from jax.experimental.pallas import tpu as pltpu
from jax.experimental.pallas import tpu_sc as plsc


@functools.cache
def _make_gather(V, D, S, B, NC, NS):
    NW = NC * NS
    L = 16
    BB = B // NW
    TS = S // 8
    DP = 2 * D
    mesh = plsc.VectorSubcoreMesh(core_axis_name="c", subcore_axis_name="s")

    @functools.partial(
        pl.kernel,
        mesh=mesh,
        compiler_params=pltpu.CompilerParams(needs_layout_passes=False),
        out_type=jax.ShapeDtypeStruct((S, D, B), jnp.float32),
        scratch_types=(
            [pltpu.VMEM((8, BB), jnp.int32)]
            + [pltpu.VMEM((BB, DP), jnp.float32) for _ in range(2)]
            + [pltpu.VMEM((D, BB), jnp.float32) for _ in range(2)]
            + [pltpu.SemaphoreType.DMA for _ in range(4)]
        ),
    )
    def k(idx_hbm, tab_hbm, out_hbm, idx_v, g0, g1, s0, s1, gs0, gs1, os0, os1):
        gbuf = (g0, g1)
        slab = (s0, s1)
        gsem = (gs0, gs1)
        osem = (os0, os1)
        wid = lax.axis_index("s") * NC + lax.axis_index("c")
        b0 = wid * BB

        def gather_start(r, sl):
            pltpu.async_copy(tab_hbm.at[idx_v.at[r]], gbuf[sl], gsem[sl])

        def gather_wait(r, sl):
            pltpu.make_async_copy(
                tab_hbm.at[idx_v.at[r]], gbuf[sl], gsem[sl]
            ).wait()

        def slab_start(s, sl):
            pltpu.async_copy(
                slab[sl], out_hbm.at[s, :, pl.ds(b0, BB)], osem[sl]
            )

        def slab_wait(s, sl):
            pltpu.make_async_copy(
                slab[sl], out_hbm.at[s, :, pl.ds(b0, BB)], osem[sl]
            ).wait()

        jrows = [jb * L + lax.iota(jnp.int32, L) for jb in range(BB // L)]

        def transpose(sl):
            def trd(d, carry):
                dcol = jnp.full((L,), d, jnp.int32)
                for jb in range(BB // L):
                    v = plsc.load_gather(gbuf[sl], [jrows[jb], dcol])
                    slab[sl][d, pl.ds(jb * L, L)] = v
                return carry

            lax.fori_loop(0, D, trd, 0)

        def tile_body(t, carry):
            pltpu.sync_copy(
                idx_hbm.at[pl.ds(t * 8, 8), pl.ds(b0, BB)], idx_v
            )
            gather_start(0, 0)
            for r in range(8):
                sl = r % 2
                if r < 7:
                    gather_start(r + 1, (r + 1) % 2)
                gather_wait(r, sl)
                s = t * 8 + r
                if r >= 2:
                    slab_wait(s - 2, sl)
                else:

                    @pl.when(t > 0)
                    def _():
                        slab_wait(s - 2, sl)

                transpose(sl)
                slab_start(s, sl)
            return carry

        lax.fori_loop(0, TS, tile_body, 0)
        slab_wait(S - 2, 0)
        slab_wait(S - 1, 1)

    return k


def kernel(input, weight):
    V, D = weight.shape
    Bt, S = input.shape
    idxt = input.T.astype(jnp.int32)
    wpad = jnp.pad(weight, ((0, 0), (0, D)))
    info = plsc.get_sparse_core_info()
    out_t = _make_gather(V, D, S, Bt, info.num_cores, info.num_subcores)(
        idxt, wpad
    )
    return out_t.transpose(2, 0, 1)


# speedup vs baseline: 1.1428x; 1.1428x over previous
import functools

import jax
import jax.numpy as jnp
from jax import lax
from jax.experimental import pallas as pl
from jax.experimental.pallas import tpu as pltpu
from jax.experimental.pallas import tpu_sc as plsc


@functools.cache
def _make_gather(V, D, S, B, NC, NS):
    NW = NC * NS
    L = 16
    BB = B // NW
    TS = S // 8
    DP = 2 * D
    mesh = plsc.VectorSubcoreMesh(core_axis_name="c", subcore_axis_name="s")

    @functools.partial(
        pl.kernel,
        mesh=mesh,
        compiler_params=pltpu.CompilerParams(needs_layout_passes=False),
        out_type=jax.ShapeDtypeStruct((S, D, B), jnp.float32),
        scratch_types=(
            [pltpu.VMEM((8, BB), jnp.int32)]
            + [pltpu.VMEM((BB, DP), jnp.float32) for _ in range(2)]
            + [pltpu.VMEM((D, BB), jnp.float32) for _ in range(2)]
            + [pltpu.SemaphoreType.DMA for _ in range(4)]
        ),
    )
    def k(idx_hbm, tab_hbm, out_hbm, idx_v, g0, g1, s0, s1, gs0, gs1, os0, os1):
        gbuf = (g0, g1)
        slab = (s0, s1)
        gsem = (gs0, gs1)
        osem = (os0, os1)
        wid = lax.axis_index("s") * NC + lax.axis_index("c")
        b0 = wid * BB

        def gather_start(r, sl):
            pltpu.async_copy(tab_hbm.at[idx_v.at[r]], gbuf[sl], gsem[sl])

        def gather_wait(r, sl):
            pltpu.make_async_copy(
                tab_hbm.at[idx_v.at[r]], gbuf[sl], gsem[sl]
            ).wait()

        def slab_start(s, sl):
            pltpu.async_copy(
                slab[sl], out_hbm.at[s, :, pl.ds(b0, BB)], osem[sl]
            )

        def slab_wait(s, sl):
            pltpu.make_async_copy(
                slab[sl], out_hbm.at[s, :, pl.ds(b0, BB)], osem[sl]
            ).wait()

        dvecs = [db * L + lax.iota(jnp.int32, L) for db in range(D // L)]

        def transpose(sl):
            def trj(jg, carry):
                for jj in range(4):
                    j = jg * 4 + jj
                    jsplat = jnp.full((L,), j, jnp.int32)
                    for db in range(D // L):
                        v = gbuf[sl][j, pl.ds(db * L, L)]
                        plsc.store_scatter(slab[sl], [dvecs[db], jsplat], v)
                return carry

            lax.fori_loop(0, BB // 4, trj, 0)

        def tile_body(t, carry):
            pltpu.sync_copy(
                idx_hbm.at[pl.ds(t * 8, 8), pl.ds(b0, BB)], idx_v
            )
            gather_start(0, 0)
            for r in range(8):
                sl = r % 2
                if r < 7:
                    gather_start(r + 1, (r + 1) % 2)
                gather_wait(r, sl)
                s = t * 8 + r
                if r >= 2:
                    slab_wait(s - 2, sl)
                else:

                    @pl.when(t > 0)
                    def _():
                        slab_wait(s - 2, sl)

                transpose(sl)
                slab_start(s, sl)
            return carry

        lax.fori_loop(0, TS, tile_body, 0)
        slab_wait(S - 2, 0)
        slab_wait(S - 1, 1)

    return k


def kernel(input, weight):
    V, D = weight.shape
    Bt, S = input.shape
    idxt = input.T.astype(jnp.int32)
    wpad = jnp.pad(weight, ((0, 0), (0, D)))
    info = plsc.get_sparse_core_info()
    out_t = _make_gather(V, D, S, Bt, info.num_cores, info.num_subcores)(
        idxt, wpad
    )
    return out_t.transpose(2, 0, 1)


# speedup vs baseline: 1.1456x; 1.0024x over previous
import functools

import jax
import jax.numpy as jnp
from jax import lax
from jax.experimental import pallas as pl
from jax.experimental.pallas import tpu as pltpu
from jax.experimental.pallas import tpu_sc as plsc


@functools.cache
def _make_gather(V, D, S, B, NC, NS):
    NW = NC * NS
    L = 16
    BB = B // NW
    TS = S // 8
    DP = 2 * D
    mesh = plsc.VectorSubcoreMesh(core_axis_name="c", subcore_axis_name="s")

    @functools.partial(
        pl.kernel,
        mesh=mesh,
        compiler_params=pltpu.CompilerParams(needs_layout_passes=False),
        out_type=jax.ShapeDtypeStruct((S, D, B), jnp.float32),
        scratch_types=(
            [pltpu.VMEM((8, BB), jnp.int32)]
            + [pltpu.VMEM((BB, DP), jnp.float32) for _ in range(2)]
            + [pltpu.VMEM((D, BB), jnp.float32) for _ in range(2)]
            + [pltpu.SemaphoreType.DMA for _ in range(4)]
        ),
    )
    def k(idx_hbm, tab_hbm, out_hbm, idx_v, g0, g1, s0, s1, gs0, gs1, os0, os1):
        gbuf = (g0, g1)
        slab = (s0, s1)
        gsem = (gs0, gs1)
        osem = (os0, os1)
        wid = lax.axis_index("s") * NC + lax.axis_index("c")
        b0 = wid * BB

        def gather_start(r, sl):
            pltpu.async_copy(tab_hbm.at[idx_v.at[r]], gbuf[sl], gsem[sl])

        def gather_wait(r, sl):
            pltpu.make_async_copy(
                tab_hbm.at[idx_v.at[r]], gbuf[sl], gsem[sl]
            ).wait()

        def slab_start(s, sl):
            pltpu.async_copy(
                slab[sl], out_hbm.at[s, :, pl.ds(b0, BB)], osem[sl]
            )

        def slab_wait(s, sl):
            pltpu.make_async_copy(
                slab[sl], out_hbm.at[s, :, pl.ds(b0, BB)], osem[sl]
            ).wait()

        dvecs = [db * L + lax.iota(jnp.int32, L) for db in range(D // L)]

        def transpose(sl):
            def trj(jg, carry):
                jbase = jg * L
                jsplat = jnp.full((L,), jbase, jnp.int32)
                for jj in range(L):
                    for db in range(D // L):
                        v = gbuf[sl][jbase + jj, pl.ds(db * L, L)]
                        plsc.store_scatter(
                            slab[sl], [dvecs[db], jsplat + jj], v
                        )
                return carry

            lax.fori_loop(0, BB // L, trj, 0)

        def tile_body(t, carry):
            pltpu.sync_copy(
                idx_hbm.at[pl.ds(t * 8, 8), pl.ds(b0, BB)], idx_v
            )
            gather_start(0, 0)
            for r in range(8):
                sl = r % 2
                if r < 7:
                    gather_start(r + 1, (r + 1) % 2)
                gather_wait(r, sl)
                s = t * 8 + r
                if r >= 2:
                    slab_wait(s - 2, sl)
                else:

                    @pl.when(t > 0)
                    def _():
                        slab_wait(s - 2, sl)

                transpose(sl)
                slab_start(s, sl)
            return carry

        lax.fori_loop(0, TS, tile_body, 0)
        slab_wait(S - 2, 0)
        slab_wait(S - 1, 1)

    return k


def kernel(input, weight):
    V, D = weight.shape
    Bt, S = input.shape
    idxt = input.T.astype(jnp.int32)
    wpad = jnp.pad(weight, ((0, 0), (0, D)))
    info = plsc.get_sparse_core_info()
    out_t = _make_gather(V, D, S, Bt, info.num_cores, info.num_subcores)(
        idxt, wpad
    )
    return out_t.transpose(2, 0, 1)


# speedup vs baseline: 1.2231x; 1.0677x over previous
import functools

import jax
import jax.numpy as jnp
from jax import lax
from jax.experimental import pallas as pl
from jax.experimental.pallas import tpu as pltpu
from jax.experimental.pallas import tpu_sc as plsc


@functools.cache
def _make_gather(V, D, S, B, NC, NS):
    NW = NC * NS
    L = 16
    BB = B // NW
    TS = S // 8
    DP = 2 * D
    mesh = plsc.VectorSubcoreMesh(core_axis_name="c", subcore_axis_name="s")

    @functools.partial(
        pl.kernel,
        mesh=mesh,
        compiler_params=pltpu.CompilerParams(needs_layout_passes=False),
        out_type=jax.ShapeDtypeStruct((S, D, B), jnp.float32),
        scratch_types=(
            [pltpu.VMEM((8, BB), jnp.int32)]
            + [pltpu.VMEM((BB, DP), jnp.float32) for _ in range(2)]
            + [pltpu.VMEM((D, BB), jnp.float32) for _ in range(2)]
            + [pltpu.SemaphoreType.DMA for _ in range(4)]
        ),
    )
    def k(idx_hbm, tab_hbm, out_hbm, idx_v, g0, g1, s0, s1, gs0, gs1, os0, os1):
        gbuf = (g0, g1)
        slab = (s0, s1)
        gsem = (gs0, gs1)
        osem = (os0, os1)
        wid = lax.axis_index("s") * NC + lax.axis_index("c")
        b0 = wid * BB

        def gather_start(r, sl):
            pltpu.async_copy(tab_hbm.at[idx_v.at[r]], gbuf[sl], gsem[sl])

        def gather_wait(r, sl):
            pltpu.make_async_copy(
                tab_hbm.at[idx_v.at[r]], gbuf[sl], gsem[sl]
            ).wait()

        def slab_start(s, sl):
            pltpu.async_copy(
                slab[sl], out_hbm.at[s, :, pl.ds(b0, BB)], osem[sl]
            )

        def slab_wait(s, sl):
            pltpu.make_async_copy(
                slab[sl], out_hbm.at[s, :, pl.ds(b0, BB)], osem[sl]
            ).wait()

        dvecs = [db * L + lax.iota(jnp.int32, L) for db in range(D // L)]

        def transpose(sl):
            @plsc.parallel_loop(0, BB // L, step=1, unroll=2)
            def trj(jg):
                jbase = jg * L
                jsplat = jnp.full((L,), jbase, jnp.int32)
                for jj in range(L):
                    vs = [
                        gbuf[sl][jbase + jj, pl.ds(db * L, L)]
                        for db in range(D // L)
                    ]
                    for db in range(D // L):
                        plsc.store_scatter(
                            slab[sl], [dvecs[db], jsplat + jj], vs[db]
                        )

        def tile_body(t, carry):
            pltpu.sync_copy(
                idx_hbm.at[pl.ds(t * 8, 8), pl.ds(b0, BB)], idx_v
            )
            gather_start(0, 0)
            for r in range(8):
                sl = r % 2
                if r < 7:
                    gather_start(r + 1, (r + 1) % 2)
                gather_wait(r, sl)
                s = t * 8 + r
                if r >= 2:
                    slab_wait(s - 2, sl)
                else:

                    @pl.when(t > 0)
                    def _():
                        slab_wait(s - 2, sl)

                transpose(sl)
                slab_start(s, sl)
            return carry

        lax.fori_loop(0, TS, tile_body, 0)
        slab_wait(S - 2, 0)
        slab_wait(S - 1, 1)

    return k


def kernel(input, weight):
    V, D = weight.shape
    Bt, S = input.shape
    idxt = input.T.astype(jnp.int32)
    wpad = jnp.pad(weight, ((0, 0), (0, D)))
    info = plsc.get_sparse_core_info()
    out_t = _make_gather(V, D, S, Bt, info.num_cores, info.num_subcores)(
        idxt, wpad
    )
    return out_t.transpose(2, 0, 1)


# speedup vs baseline: 1.4744x; 1.2054x over previous
import functools

import jax
import jax.numpy as jnp
from jax import lax
from jax.experimental import pallas as pl
from jax.experimental.pallas import tpu as pltpu
from jax.experimental.pallas import tpu_sc as plsc

_CHUNK = 256
_NBUF = 4


@functools.cache
def _make_gather(V, D, B, NC, NS):
    NW = NC * NS
    b_per_w = B // NW
    C = _CHUNK
    nbuf = _NBUF
    nchunks = b_per_w // C
    ngroups = nchunks // nbuf
    assert nchunks % nbuf == 0
    mesh = plsc.VectorSubcoreMesh(core_axis_name="c", subcore_axis_name="s")

    @functools.partial(
        pl.kernel,
        mesh=mesh,
        compiler_params=pltpu.CompilerParams(use_tc_tiling_on_sc=False),
        out_type=jax.ShapeDtypeStruct((B, D), jnp.float32),
        scratch_types=(
            [pltpu.VMEM((b_per_w,), jnp.int32)]
            + [pltpu.VMEM((C, D), jnp.float32) for _ in range(nbuf)]
            + [pltpu.SemaphoreType.DMA for _ in range(2 * nbuf)]
        ),
    )
    def k(idx_hbm, table_hbm, out_hbm, idx_v, *bufs_and_sems):
        bufs = bufs_and_sems[:nbuf]
        gsem = bufs_and_sems[nbuf : 2 * nbuf]
        wsem = bufs_and_sems[2 * nbuf :]
        wid = lax.axis_index("s") * NC + lax.axis_index("c")
        base0 = wid * b_per_w

        pltpu.sync_copy(idx_hbm.at[pl.ds(base0, b_per_w)], idx_v)

        def gather_start(i, b):
            idx_slice = idx_v.at[pl.ds(i * C, C)]
            pltpu.async_copy(table_hbm.at[idx_slice], bufs[b], gsem[b])

        def gather_wait(i, b):
            idx_slice = idx_v.at[pl.ds(i * C, C)]
            pltpu.make_async_copy(table_hbm.at[idx_slice], bufs[b], gsem[b]).wait()

        def wb_start(i, b):
            pltpu.async_copy(bufs[b], out_hbm.at[pl.ds(base0 + i * C, C)], wsem[b])

        def wb_wait(i, b):
            pltpu.make_async_copy(
                bufs[b], out_hbm.at[pl.ds(base0 + i * C, C)], wsem[b]
            ).wait()

        for b in range(nbuf):
            gather_start(b, b)

        def body(g, carry):
            for b in range(nbuf):
                i = g * nbuf + b
                gather_wait(i, b)
                wb_start(i, b)
                wb_wait(i, b)
                gather_start(i + nbuf, b)
            return carry

        lax.fori_loop(0, ngroups - 1, body, 0)

        for b in range(nbuf):
            i = (ngroups - 1) * nbuf + b
            gather_wait(i, b)
            wb_start(i, b)
        for b in range(nbuf):
            i = (ngroups - 1) * nbuf + b
            wb_wait(i, b)

    return k


def kernel(input, weight):
    V, D = weight.shape
    idx = input.reshape(-1).astype(jnp.int32)
    B = idx.shape[0]
    info = plsc.get_sparse_core_info()
    out = _make_gather(V, D, B, info.num_cores, info.num_subcores)(idx, weight)
    return out.reshape(input.shape + (D,))


# speedup vs baseline: 1.5549x; 1.0546x over previous
import functools

import jax
import jax.numpy as jnp
from jax import lax
from jax.experimental import pallas as pl
from jax.experimental.pallas import tpu as pltpu
from jax.experimental.pallas import tpu_sc as plsc

_CHUNK = 256
_NBUF = 4


@functools.cache
def _make_gather(V, D, B, NC, NS):
    NW = NC * NS
    b_per_w = B // NW
    C = _CHUNK
    nbuf = _NBUF
    nchunks = b_per_w // C
    ngroups = nchunks // nbuf
    assert nchunks % nbuf == 0
    mesh = plsc.VectorSubcoreMesh(core_axis_name="c", subcore_axis_name="s")

    @functools.partial(
        pl.kernel,
        mesh=mesh,
        compiler_params=pltpu.CompilerParams(use_tc_tiling_on_sc=False),
        out_type=jax.ShapeDtypeStruct((B, D), jnp.float32),
        scratch_types=(
            [pltpu.VMEM((b_per_w,), jnp.int32)]
            + [pltpu.VMEM((C, D), jnp.float32) for _ in range(nbuf)]
            + [pltpu.SemaphoreType.DMA for _ in range(2 * nbuf)]
        ),
    )
    def k(idx_hbm, table_hbm, out_hbm, idx_v, *bufs_and_sems):
        bufs = bufs_and_sems[:nbuf]
        gsem = bufs_and_sems[nbuf : 2 * nbuf]
        wsem = bufs_and_sems[2 * nbuf :]
        wid = lax.axis_index("s") * NC + lax.axis_index("c")
        base0 = wid * b_per_w

        pltpu.sync_copy(idx_hbm.at[pl.ds(base0, b_per_w)], idx_v)

        def gather_start(i, b):
            idx_slice = idx_v.at[pl.ds(i * C, C)]
            pltpu.async_copy(table_hbm.at[idx_slice], bufs[b], gsem[b])

        def gather_wait(i, b):
            idx_slice = idx_v.at[pl.ds(i * C, C)]
            pltpu.make_async_copy(table_hbm.at[idx_slice], bufs[b], gsem[b]).wait()

        def wb_start(i, b):
            pltpu.async_copy(bufs[b], out_hbm.at[pl.ds(base0 + i * C, C)], wsem[b])

        def wb_wait(i, b):
            pltpu.make_async_copy(
                bufs[b], out_hbm.at[pl.ds(base0 + i * C, C)], wsem[b]
            ).wait()

        for b in range(nbuf):
            gather_start(b, b)

        def body(g, carry):
            for b in range(nbuf):
                i = g * nbuf + b
                gather_wait(i, b)
                wb_start(i, b)
                wb_wait(i, b)
                gather_start(i + nbuf, b)
            return carry

        lax.fori_loop(0, ngroups - 1, body, 0)

        for b in range(nbuf):
            i = (ngroups - 1) * nbuf + b
            gather_wait(i, b)
            wb_start(i, b)
        for b in range(nbuf):
            i = (ngroups - 1) * nbuf + b
            wb_wait(i, b)

    return k


def kernel(input, weight):
    V, D = weight.shape
    idx = (input.reshape(-1) * 2).astype(jnp.int32)
    B = idx.shape[0]
    wpad = jnp.pad(weight, ((0, 0), (0, D))).reshape(2 * V, D)
    info = plsc.get_sparse_core_info()
    out = _make_gather(2 * V, D, B, info.num_cores, info.num_subcores)(idx, wpad)
    return out.reshape(input.shape + (D,))


# speedup vs baseline: 1.8109x; 1.1646x over previous
import functools

import jax
import jax.numpy as jnp
from jax import lax
from jax.experimental import pallas as pl
from jax.experimental.pallas import tpu as pltpu
from jax.experimental.pallas import tpu_sc as plsc

_CHUNK = 128
_NBUF = 4


@functools.cache
def _make_gather(V, D, B, NC, NS):
    NW = NC * NS
    b_per_w = B // NW
    C = _CHUNK
    nbuf = _NBUF
    nchunks = b_per_w // C
    ngroups = nchunks // nbuf
    assert nchunks % nbuf == 0
    mesh = plsc.VectorSubcoreMesh(core_axis_name="c", subcore_axis_name="s")

    @functools.partial(
        pl.kernel,
        mesh=mesh,
        compiler_params=pltpu.CompilerParams(use_tc_tiling_on_sc=False),
        out_type=jax.ShapeDtypeStruct((B, D), jnp.float32),
        scratch_types=(
            [pltpu.VMEM((b_per_w,), jnp.int32)]
            + [pltpu.VMEM((C, D), jnp.float32) for _ in range(nbuf)]
            + [pltpu.SemaphoreType.DMA for _ in range(2 * nbuf)]
        ),
    )
    def k(idx_hbm, table_hbm, out_hbm, idx_v, *bufs_and_sems):
        bufs = bufs_and_sems[:nbuf]
        gsem = bufs_and_sems[nbuf : 2 * nbuf]
        wsem = bufs_and_sems[2 * nbuf :]
        wid = lax.axis_index("s") * NC + lax.axis_index("c")
        base0 = wid * b_per_w

        pltpu.sync_copy(idx_hbm.at[pl.ds(base0, b_per_w)], idx_v)

        def gather_start(i, b):
            idx_slice = idx_v.at[pl.ds(i * C, C)]
            pltpu.async_copy(table_hbm.at[idx_slice], bufs[b], gsem[b])

        def gather_wait(i, b):
            idx_slice = idx_v.at[pl.ds(i * C, C)]
            pltpu.make_async_copy(table_hbm.at[idx_slice], bufs[b], gsem[b]).wait()

        def wb_start(i, b):
            pltpu.async_copy(bufs[b], out_hbm.at[pl.ds(base0 + i * C, C)], wsem[b])

        def wb_wait(i, b):
            pltpu.make_async_copy(
                bufs[b], out_hbm.at[pl.ds(base0 + i * C, C)], wsem[b]
            ).wait()

        for b in range(nbuf):
            gather_start(b, b)

        def body(g, carry):
            for b in range(nbuf):
                i = g * nbuf + b
                gather_wait(i, b)
                wb_start(i, b)
                wb_wait(i, b)
                gather_start(i + nbuf, b)
            return carry

        lax.fori_loop(0, ngroups - 1, body, 0)

        for b in range(nbuf):
            i = (ngroups - 1) * nbuf + b
            gather_wait(i, b)
            wb_start(i, b)
        for b in range(nbuf):
            i = (ngroups - 1) * nbuf + b
            wb_wait(i, b)

    return k


def kernel(input, weight):
    V, D = weight.shape
    idx = input.reshape(-1).astype(jnp.int32)
    B = idx.shape[0]
    wpad = jnp.pad(weight, ((0, 0), (0, D)))
    info = plsc.get_sparse_core_info()
    out = _make_gather(V, 2 * D, B, info.num_cores, info.num_subcores)(idx, wpad)
    return out.reshape(input.shape + (2 * D,))[:, :, :D]
